# baseline (speedup 1.0000x reference)
import functools

import jax
import jax.numpy as jnp
from jax import lax
from jax.experimental import pallas as pl
from jax.experimental.pallas import tpu as pltpu
from jax.experimental.pallas import tpu_sc as plsc

B = 16384
NCAT = 26
NNUM = 13
EMB = 32
CARD = 40000
NROWS = NCAT + NNUM

NC = 2
NS = 16
NW = NC * NS
PB = B // NW
TJW = PB // 128
NVB = PB // 16


def _body(cat_hbm, num_hbm, table_hbm, dir_hbm, anc_hbm, out_hbm,
          idx0, idx1, stage0, stage1, obuf0, obuf1, nobuf,
          nbuf, dir_v, anc_v, gsem0, gsem1, osem0, osem1, nsem):
    wid = lax.axis_index("s") * NC + lax.axis_index("c")
    bw = wid * PB

    pltpu.sync_copy(dir_hbm, dir_v)
    pltpu.sync_copy(anc_hbm, anc_v)

    lane = lax.iota(jnp.int32, 16)
    lane32 = lane * EMB

    idx = (idx0, idx1)
    stage = (stage0, stage1)
    obuf = (obuf0, obuf1)
    gsem = (gsem0, gsem1)
    osem = (osem0, osem1)

    def load_and_fire(j, h):
        pltpu.sync_copy(cat_hbm.at[j, pl.ds(bw, PB)], idx[h])
        off = j * CARD

        def add_off(v, c):
            idx[h][pl.ds(v * 16, 16)] = idx[h][pl.ds(v * 16, 16)] + off
            return c
        lax.fori_loop(0, NVB, add_off, None)
        pltpu.async_copy(table_hbm.at[idx[h]], stage[h], gsem[h])

    def process(j, h, first):
        pltpu.make_async_copy(table_hbm.at[idx[h]], stage[h], gsem[h]).wait()

        @pl.when(jnp.logical_not(first))
        def _():
            pltpu.make_async_copy(
                obuf[h], out_hbm.at[j, :, pl.ds(wid * TJW, TJW)],
                osem[h]).wait()

        def transpose_blk(g, c):
            ti = g // TJW
            tjl = lax.rem(g, jnp.int32(TJW))
            d0 = ti * 8
            for v in range(8):
                bidx = tjl * 128 + v * 16 + lane
                for di in range(8):
                    didx = jnp.full((16,), d0 + di, jnp.int32)
                    val = plsc.load_gather(stage[h], [bidx, didx])
                    obuf[h][ti, tjl, di, pl.ds(v * 16, 16)] = val
            return c
        lax.fori_loop(0, 4 * TJW, transpose_blk, None)

        pltpu.async_copy(obuf[h], out_hbm.at[j, :, pl.ds(wid * TJW, TJW)],
                         osem[h])

    load_and_fire(0, 0)

    def field_pair(g, c):
        j0 = g * 2
        load_and_fire(j0 + 1, 1)
        process(j0, 0, g == 0)

        @pl.when(g < NCAT // 2 - 1)
        def _():
            load_and_fire(j0 + 2, 0)
        process(j0 + 1, 1, g == 0)
        return c
    lax.fori_loop(0, NCAT // 2, field_pair, None)

    for h in (0, 1):
        pltpu.make_async_copy(
            obuf[h], out_hbm.at[NCAT - 2 + h, :, pl.ds(wid * TJW, TJW)],
            osem[h]).wait()

    for f in range(NNUM):
        pltpu.sync_copy(num_hbm.at[f, pl.ds(bw, PB)], nbuf)
        dv = (dir_v[f, pl.ds(0, 16)], dir_v[f, pl.ds(16, 16)])
        av = (anc_v[f, pl.ds(0, 16)], anc_v[f, pl.ds(16, 16)])

        if f >= 1:
            pltpu.make_async_copy(
                nobuf, out_hbm.at[NCAT + f - 1, :, pl.ds(wid * TJW, TJW)],
                nsem).wait()

        def num_blk(bb, c, dv=dv, av=av):
            tjl = bb // 8
            v = lax.rem(bb, jnp.int32(8))
            nb = nbuf[pl.ds(bb * 16, 16)]
            for d in range(EMB):
                s = dv[d // 16][d % 16]
                a = av[d // 16][d % 16]
                nobuf[d // 8, tjl, d % 8, pl.ds(v * 16, 16)] = nb * s + a
            return c
        lax.fori_loop(0, NVB, num_blk, None)

        pltpu.async_copy(nobuf, out_hbm.at[NCAT + f, :, pl.ds(wid * TJW, TJW)],
                         nsem)

    pltpu.make_async_copy(
        nobuf, out_hbm.at[NROWS - 1, :, pl.ds(wid * TJW, TJW)], nsem).wait()


@jax.jit
def _run(cat_t, num_t, table, direction, anchor):
    mesh = plsc.VectorSubcoreMesh(core_axis_name="c", subcore_axis_name="s")
    fn = pl.kernel(
        _body,
        out_type=jax.ShapeDtypeStruct((NROWS, 4, 128, 8, 128), jnp.float32),
        mesh=mesh,
        scratch_types=[
            pltpu.VMEM((PB,), jnp.int32),
            pltpu.VMEM((PB,), jnp.int32),
            pltpu.VMEM((PB, EMB), jnp.float32),
            pltpu.VMEM((PB, EMB), jnp.float32),
            pltpu.VMEM((4, TJW, 8, 128), jnp.float32),
            pltpu.VMEM((4, TJW, 8, 128), jnp.float32),
            pltpu.VMEM((4, TJW, 8, 128), jnp.float32),
            pltpu.VMEM((PB,), jnp.float32),
            pltpu.VMEM((NNUM, EMB), jnp.float32),
            pltpu.VMEM((NNUM, EMB), jnp.float32),
            pltpu.SemaphoreType.DMA,
            pltpu.SemaphoreType.DMA,
            pltpu.SemaphoreType.DMA,
            pltpu.SemaphoreType.DMA,
            pltpu.SemaphoreType.DMA,
        ],
        compiler_params=pltpu.CompilerParams(use_tc_tiling_on_sc=False,
                                             needs_layout_passes=False),
    )
    return fn(cat_t, num_t, table, direction, anchor)


def kernel(cat_features, num_features, table, direction, anchor):
    cat_t = cat_features.astype(jnp.int32).T
    num_t = num_features.T
    out5 = _run(cat_t, num_t, table, direction, anchor)
    return out5.transpose(2, 4, 0, 1, 3).reshape(B, NROWS, EMB)


# speedup vs baseline: 1.0263x; 1.0263x over previous
import functools

import jax
import jax.numpy as jnp
from jax import lax
from jax.experimental import pallas as pl
from jax.experimental.pallas import tpu as pltpu
from jax.experimental.pallas import tpu_sc as plsc

B = 16384
NCAT = 26
NNUM = 13
EMB = 32
CARD = 40000
NROWS = NCAT + NNUM

NC = 2
NS = 16
NW = NC * NS
PB = B // NW
TJW = PB // 128
NVB = PB // 16


def _body(cat_hbm, num_hbm, table_hbm, dir_hbm, anc_hbm, out_hbm,
          idxall, numall, stage0, stage1, stage2, stage3, obuf0, obuf1,
          dir_v, anc_v, gsem0, gsem1, gsem2, gsem3, osem0, osem1):
    wid = lax.axis_index("s") * NC + lax.axis_index("c")
    bw = wid * PB

    pltpu.sync_copy(dir_hbm, dir_v)
    pltpu.sync_copy(anc_hbm, anc_v)

    pltpu.sync_copy(cat_hbm.at[:, pl.ds(bw, PB)], idxall)
    pltpu.sync_copy(num_hbm.at[:, pl.ds(bw, PB)], numall)

    lane = lax.iota(jnp.int32, 16)

    stage = (stage0, stage1, stage2, stage3)
    obuf = (obuf0, obuf1)
    gsem = (gsem0, gsem1, gsem2, gsem3)
    osem = (osem0, osem1)
    NSTG = 4

    def add_off(w, c):
        j = w // (NVB // 4)
        v = lax.rem(w, jnp.int32(NVB // 4))
        for u in range(4):
            p = (v * 4 + u) * 16
            idxall[j, pl.ds(p, 16)] = idxall[j, pl.ds(p, 16)] + j * CARD
        return c
    lax.fori_loop(0, NCAT * (NVB // 4), add_off, None)

    def fire(j, hs):
        pltpu.async_copy(table_hbm.at[idxall.at[j]], stage[hs], gsem[hs])

    def process(j, hs, ho, first):
        pltpu.make_async_copy(table_hbm.at[idxall.at[j]], stage[hs],
                              gsem[hs]).wait()

        @pl.when(jnp.logical_not(first))
        def _():
            pltpu.make_async_copy(
                obuf[ho], out_hbm.at[j, :, pl.ds(wid * TJW, TJW)],
                osem[ho]).wait()

        def transpose_blk(g, c):
            ti = g // TJW
            tjl = lax.rem(g, jnp.int32(TJW))
            d0 = ti * 8
            for v in range(8):
                bidx = tjl * 128 + v * 16 + lane
                for di in range(8):
                    didx = jnp.full((16,), d0 + di, jnp.int32)
                    val = plsc.load_gather(stage[hs], [bidx, didx])
                    obuf[ho][ti, tjl, di, pl.ds(v * 16, 16)] = val
            return c
        lax.fori_loop(0, 4 * TJW, transpose_blk, None)

        pltpu.async_copy(obuf[ho], out_hbm.at[j, :, pl.ds(wid * TJW, TJW)],
                         osem[ho])

    for p in range(NSTG):
        fire(p, p)

    def field_quad(g, c):
        for p in range(NSTG):
            j = g * NSTG + p

            @pl.when(j < NCAT)
            def _(j=j, p=p):
                process(j, p, p % 2,
                        (g == 0) if p < 2 else jnp.bool_(False))

                @pl.when(j + NSTG < NCAT)
                def _():
                    fire(j + NSTG, p)
        return c
    lax.fori_loop(0, (NCAT + NSTG - 1) // NSTG, field_quad, None)

    for ho in (0, 1):
        pltpu.make_async_copy(
            obuf[ho], out_hbm.at[NCAT - 2 + ho, :, pl.ds(wid * TJW, TJW)],
            osem[ho]).wait()

    for f in range(NNUM):
        ho = f % 2
        dv = (dir_v[f, pl.ds(0, 16)], dir_v[f, pl.ds(16, 16)])
        av = (anc_v[f, pl.ds(0, 16)], anc_v[f, pl.ds(16, 16)])

        if f >= 2:
            pltpu.make_async_copy(
                obuf[ho], out_hbm.at[NCAT + f - 2, :, pl.ds(wid * TJW, TJW)],
                osem[ho]).wait()

        def num_blk(bb, c, dv=dv, av=av, ho=ho, f=f):
            tjl = bb // 8
            v = lax.rem(bb, jnp.int32(8))
            nb = numall[f, pl.ds(bb * 16, 16)]
            for d in range(EMB):
                s = dv[d // 16][d % 16]
                a = av[d // 16][d % 16]
                obuf[ho][d // 8, tjl, d % 8, pl.ds(v * 16, 16)] = nb * s + a
            return c
        lax.fori_loop(0, NVB, num_blk, None)

        pltpu.async_copy(obuf[ho],
                         out_hbm.at[NCAT + f, :, pl.ds(wid * TJW, TJW)],
                         osem[ho])

    for f in (NNUM - 2, NNUM - 1):
        pltpu.make_async_copy(
            obuf[f % 2], out_hbm.at[NCAT + f, :, pl.ds(wid * TJW, TJW)],
            osem[f % 2]).wait()


@jax.jit
def _run(cat_t, num_t, table, direction, anchor):
    mesh = plsc.VectorSubcoreMesh(core_axis_name="c", subcore_axis_name="s")
    fn = pl.kernel(
        _body,
        out_type=jax.ShapeDtypeStruct((NROWS, 4, 128, 8, 128), jnp.float32),
        mesh=mesh,
        scratch_types=[
            pltpu.VMEM((NCAT, PB), jnp.int32),
            pltpu.VMEM((NNUM, PB), jnp.float32),
            pltpu.VMEM((PB, EMB), jnp.float32),
            pltpu.VMEM((PB, EMB), jnp.float32),
            pltpu.VMEM((PB, EMB), jnp.float32),
            pltpu.VMEM((PB, EMB), jnp.float32),
            pltpu.VMEM((4, TJW, 8, 128), jnp.float32),
            pltpu.VMEM((4, TJW, 8, 128), jnp.float32),
            pltpu.VMEM((NNUM, EMB), jnp.float32),
            pltpu.VMEM((NNUM, EMB), jnp.float32),
            pltpu.SemaphoreType.DMA,
            pltpu.SemaphoreType.DMA,
            pltpu.SemaphoreType.DMA,
            pltpu.SemaphoreType.DMA,
            pltpu.SemaphoreType.DMA,
            pltpu.SemaphoreType.DMA,
        ],
        compiler_params=pltpu.CompilerParams(use_tc_tiling_on_sc=False,
                                             needs_layout_passes=False),
    )
    return fn(cat_t, num_t, table, direction, anchor)


def kernel(cat_features, num_features, table, direction, anchor):
    cat_t = cat_features.astype(jnp.int32).T
    num_t = num_features.T
    out5 = _run(cat_t, num_t, table, direction, anchor)
    return out5.transpose(2, 4, 0, 1, 3).reshape(B, NROWS, EMB)


# speedup vs baseline: 1.0286x; 1.0022x over previous
import functools

import jax
import jax.numpy as jnp
from jax import lax
from jax.experimental import pallas as pl
from jax.experimental.pallas import tpu as pltpu
from jax.experimental.pallas import tpu_sc as plsc

B = 16384
NCAT = 26
NNUM = 13
EMB = 32
CARD = 40000
NROWS = NCAT + NNUM

NC = 2
NS = 16
NW = NC * NS
PB = B // NW
TJW = PB // 128
NVB = PB // 16


def _body(cat_hbm, num_hbm, table_hbm, dir_hbm, anc_hbm, out_hbm,
          idxall, numall, stage0, stage1, stage2, stage3, obuf0, obuf1,
          dir_v, anc_v, gsem0, gsem1, gsem2, gsem3, osem0, osem1):
    wid = lax.axis_index("s") * NC + lax.axis_index("c")
    bw = wid * PB

    pltpu.sync_copy(dir_hbm, dir_v)
    pltpu.sync_copy(anc_hbm, anc_v)

    pltpu.sync_copy(cat_hbm.at[:, pl.ds(bw, PB)], idxall)
    pltpu.sync_copy(num_hbm.at[:, pl.ds(bw, PB)], numall)

    lane = lax.iota(jnp.int32, 16)

    stage = (stage0, stage1, stage2, stage3)
    obuf = (obuf0, obuf1)
    gsem = (gsem0, gsem1, gsem2, gsem3)
    osem = (osem0, osem1)
    NSTG = 4

    def add_off(w, c):
        j = w // (NVB // 4)
        v = lax.rem(w, jnp.int32(NVB // 4))
        for u in range(4):
            p = (v * 4 + u) * 16
            idxall[j, pl.ds(p, 16)] = idxall[j, pl.ds(p, 16)] + j * CARD
        return c
    lax.fori_loop(0, NCAT * (NVB // 4), add_off, None)

    def fire(j, hs):
        pltpu.async_copy(table_hbm.at[idxall.at[j]], stage[hs], gsem[hs])

    def process(j, hs, ho, first):
        pltpu.make_async_copy(table_hbm.at[idxall.at[j]], stage[hs],
                              gsem[hs]).wait()

        @pl.when(jnp.logical_not(first))
        def _():
            pltpu.make_async_copy(
                obuf[ho], out_hbm.at[j, :, pl.ds(wid * TJW, TJW)],
                osem[ho]).wait()

        def transpose_blk(g, c):
            ti = g // TJW
            tjl = lax.rem(g, jnp.int32(TJW))
            d0 = ti * 8
            for v in range(8):
                bidx = tjl * 128 + v * 16 + lane
                for di in range(8):
                    didx = jnp.full((16,), d0 + di, jnp.int32)
                    val = plsc.load_gather(stage[hs], [bidx, didx])
                    obuf[ho][ti, tjl, di, pl.ds(v * 16, 16)] = val
            return c
        lax.fori_loop(0, 4 * TJW, transpose_blk, None)

        pltpu.async_copy(obuf[ho], out_hbm.at[j, :, pl.ds(wid * TJW, TJW)],
                         osem[ho])

    for p in range(NSTG):
        fire(p, p)

    for f in range(NNUM):
        ho = f % 2
        dv = (dir_v[f, pl.ds(0, 16)], dir_v[f, pl.ds(16, 16)])
        av = (anc_v[f, pl.ds(0, 16)], anc_v[f, pl.ds(16, 16)])

        if f >= 2:
            pltpu.make_async_copy(
                obuf[ho], out_hbm.at[NCAT + f - 2, :, pl.ds(wid * TJW, TJW)],
                osem[ho]).wait()

        def num_blk(bb, c, dv=dv, av=av, ho=ho, f=f):
            tjl = bb // 8
            v = lax.rem(bb, jnp.int32(8))
            nb = numall[f, pl.ds(bb * 16, 16)]
            for d in range(EMB):
                s = dv[d // 16][d % 16]
                a = av[d // 16][d % 16]
                obuf[ho][d // 8, tjl, d % 8, pl.ds(v * 16, 16)] = nb * s + a
            return c
        lax.fori_loop(0, NVB, num_blk, None)

        pltpu.async_copy(obuf[ho],
                         out_hbm.at[NCAT + f, :, pl.ds(wid * TJW, TJW)],
                         osem[ho])

    for f in (NNUM - 2, NNUM - 1):
        pltpu.make_async_copy(
            obuf[f % 2], out_hbm.at[NCAT + f, :, pl.ds(wid * TJW, TJW)],
            osem[f % 2]).wait()


    def field_quad(g, c):
        for p in range(NSTG):
            j = g * NSTG + p

            @pl.when(j < NCAT)
            def _(j=j, p=p):
                process(j, p, p % 2,
                        (g == 0) if p < 2 else jnp.bool_(False))

                @pl.when(j + NSTG < NCAT)
                def _():
                    fire(j + NSTG, p)
        return c
    lax.fori_loop(0, (NCAT + NSTG - 1) // NSTG, field_quad, None)

    for ho in (0, 1):
        pltpu.make_async_copy(
            obuf[ho], out_hbm.at[NCAT - 2 + ho, :, pl.ds(wid * TJW, TJW)],
            osem[ho]).wait()


@jax.jit
def _run(cat_t, num_t, table, direction, anchor):
    mesh = plsc.VectorSubcoreMesh(core_axis_name="c", subcore_axis_name="s")
    fn = pl.kernel(
        _body,
        out_type=jax.ShapeDtypeStruct((NROWS, 4, 128, 8, 128), jnp.float32),
        mesh=mesh,
        scratch_types=[
            pltpu.VMEM((NCAT, PB), jnp.int32),
            pltpu.VMEM((NNUM, PB), jnp.float32),
            pltpu.VMEM((PB, EMB), jnp.float32),
            pltpu.VMEM((PB, EMB), jnp.float32),
            pltpu.VMEM((PB, EMB), jnp.float32),
            pltpu.VMEM((PB, EMB), jnp.float32),
            pltpu.VMEM((4, TJW, 8, 128), jnp.float32),
            pltpu.VMEM((4, TJW, 8, 128), jnp.float32),
            pltpu.VMEM((NNUM, EMB), jnp.float32),
            pltpu.VMEM((NNUM, EMB), jnp.float32),
            pltpu.SemaphoreType.DMA,
            pltpu.SemaphoreType.DMA,
            pltpu.SemaphoreType.DMA,
            pltpu.SemaphoreType.DMA,
            pltpu.SemaphoreType.DMA,
            pltpu.SemaphoreType.DMA,
        ],
        compiler_params=pltpu.CompilerParams(use_tc_tiling_on_sc=False,
                                             needs_layout_passes=False),
    )
    return fn(cat_t, num_t, table, direction, anchor)


def kernel(cat_features, num_features, table, direction, anchor):
    cat_t = cat_features.astype(jnp.int32).T
    num_t = num_features.T
    out5 = _run(cat_t, num_t, table, direction, anchor)
    return out5.transpose(2, 4, 0, 1, 3).reshape(B, NROWS, EMB)
